# Initial kernel scaffold; baseline (speedup 1.0000x reference)
import jax
import jax.numpy as jnp
from jax.experimental import pallas as pl


def kernel(x, mask_sites):
    raise NotImplementedError("write your pallas kernel here")


# speedup vs baseline: 1.0101x; 1.0101x over previous
import jax
import jax.numpy as jnp
from jax.experimental import pallas as pl

C, H, W = 192, 512, 512
N_SITES = 181
N_PAD = 184
C_BLK = 8


def _mask_mul_kernel(idx_ref, x_ref, o_ref):
    idx = idx_ref[...]
    cols = jax.lax.broadcasted_iota(jnp.int32, (N_PAD, W), 1)
    hit = jnp.any(cols == idx, axis=0, keepdims=True)
    keep = jnp.where(hit, 0.0, 1.0)
    o_ref[...] = x_ref[...] * keep[None]


def kernel(x, mask_sites):
    sites = jnp.full((N_PAD, 1), W, dtype=jnp.int32)
    sites = sites.at[:N_SITES, 0].set(mask_sites.astype(jnp.int32))
    grid = (C // C_BLK,)
    return pl.pallas_call(
        _mask_mul_kernel,
        grid=grid,
        in_specs=[
            pl.BlockSpec((N_PAD, 1), lambda i: (0, 0)),
            pl.BlockSpec((C_BLK, H, W), lambda i: (i, 0, 0)),
        ],
        out_specs=pl.BlockSpec((C_BLK, H, W), lambda i: (i, 0, 0)),
        out_shape=jax.ShapeDtypeStruct((C, H, W), jnp.float32),
    )(sites, x)


# speedup vs baseline: 1.0118x; 1.0017x over previous
import jax
import jax.numpy as jnp
from jax.experimental import pallas as pl

C, H, W = 192, 512, 512
N_SITES = 181
N_PAD = 184
C_BLK = 12


def _mask_mul_kernel(idx_ref, x_ref, o_ref):
    idx = idx_ref[...]
    cols = jax.lax.broadcasted_iota(jnp.int32, (N_PAD, W), 1)
    hit = jnp.any(cols == idx, axis=0, keepdims=True)
    keep = jnp.where(hit, 0.0, 1.0)
    o_ref[...] = x_ref[...] * keep[None]


def kernel(x, mask_sites):
    sites = jnp.full((N_PAD, 1), W, dtype=jnp.int32)
    sites = sites.at[:N_SITES, 0].set(mask_sites.astype(jnp.int32))
    grid = (C // C_BLK,)
    return pl.pallas_call(
        _mask_mul_kernel,
        grid=grid,
        in_specs=[
            pl.BlockSpec((N_PAD, 1), lambda i: (0, 0)),
            pl.BlockSpec((C_BLK, H, W), lambda i: (i, 0, 0)),
        ],
        out_specs=pl.BlockSpec((C_BLK, H, W), lambda i: (i, 0, 0)),
        out_shape=jax.ShapeDtypeStruct((C, H, W), jnp.float32),
    )(sites, x)
